# baseline (speedup 1.0000x reference)
import functools

import jax
import jax.numpy as jnp
from jax import lax
from jax.experimental import pallas as pl
from jax.experimental.pallas import tpu as pltpu
from jax.experimental.pallas import tpu_sc as plsc

B = 16384
GMF_D = 16
MLP_D = 32

NC = 2
NS = 16
NW = NC * NS
BPW = B // NW
IDX_CHUNK = 128
NCHUNK = BPW // IDX_CHUNK


def _sc_gather_body(u_hbm, i_hbm, ugt, igt, umt, imt,
                    ug_out, ig_out, um_out, im_out,
                    idx_u, idx_i, ug_v, ig_v, um_v, im_v, sem):
  wid = lax.axis_index("s") * NC + lax.axis_index("c")
  base = wid * BPW
  row0 = wid * NCHUNK
  pltpu.sync_copy(u_hbm.at[pl.ds(row0, NCHUNK), :], idx_u)
  pltpu.sync_copy(i_hbm.at[pl.ds(row0, NCHUNK), :], idx_i)
  copies = []
  for j in range(NCHUNK):
    dst = pl.ds(j * IDX_CHUNK, IDX_CHUNK)
    copies.append(pltpu.async_copy(ugt.at[idx_u.at[j]], ug_v.at[dst], sem))
    copies.append(pltpu.async_copy(igt.at[idx_i.at[j]], ig_v.at[dst], sem))
    copies.append(pltpu.async_copy(umt.at[idx_u.at[j]], um_v.at[dst], sem))
    copies.append(pltpu.async_copy(imt.at[idx_i.at[j]], im_v.at[dst], sem))
  for c in copies:
    c.wait()
  pltpu.sync_copy(ug_v, ug_out.at[pl.ds(base, BPW), :])
  pltpu.sync_copy(ig_v, ig_out.at[pl.ds(base, BPW), :])
  pltpu.sync_copy(um_v, um_out.at[pl.ds(base, BPW), :])
  pltpu.sync_copy(im_v, im_out.at[pl.ds(base, BPW), :])


_sc_gather = pl.kernel(
    _sc_gather_body,
    out_type=(
        jax.ShapeDtypeStruct((B, GMF_D), jnp.float32),
        jax.ShapeDtypeStruct((B, GMF_D), jnp.float32),
        jax.ShapeDtypeStruct((B, MLP_D), jnp.float32),
        jax.ShapeDtypeStruct((B, MLP_D), jnp.float32),
    ),
    mesh=plsc.VectorSubcoreMesh(core_axis_name="c", subcore_axis_name="s"),
    scratch_types=[
        pltpu.VMEM((NCHUNK, IDX_CHUNK), jnp.int32),
        pltpu.VMEM((NCHUNK, IDX_CHUNK), jnp.int32),
        pltpu.VMEM((BPW, GMF_D), jnp.float32),
        pltpu.VMEM((BPW, GMF_D), jnp.float32),
        pltpu.VMEM((BPW, MLP_D), jnp.float32),
        pltpu.VMEM((BPW, MLP_D), jnp.float32),
        pltpu.SemaphoreType.DMA,
    ],
    compiler_params=pltpu.CompilerParams(use_tc_tiling_on_sc=False),
)


BLK = 1024


def _tc_mlp_body(ug_ref, ig_ref, um_ref, im_ref,
                 w0_ref, b0_ref, w1_ref, b1_ref, wfc_ref, bfc_ref, o_ref):
  x = jnp.concatenate([um_ref[...], im_ref[...]], axis=1)
  h = jnp.maximum(jnp.dot(x, w0_ref[...]) + b0_ref[...], 0.0)
  m = jnp.maximum(jnp.dot(h, w1_ref[...]) + b1_ref[...], 0.0)
  g = ug_ref[...] * ig_ref[...]
  z = jnp.concatenate([g, m], axis=1)
  logit = jnp.dot(z, wfc_ref[...]) + bfc_ref[...]
  o_ref[...] = jax.nn.sigmoid(logit)


def _tc_mlp(ug, ig, um, im, w0, b0, w1, b1, wfc, bfc):
  grid = (B // BLK,)
  row = lambda b: (b, 0)
  rep = lambda b: (0, 0)
  return pl.pallas_call(
      _tc_mlp_body,
      grid=grid,
      in_specs=[
          pl.BlockSpec((BLK, GMF_D), row),
          pl.BlockSpec((BLK, GMF_D), row),
          pl.BlockSpec((BLK, MLP_D), row),
          pl.BlockSpec((BLK, MLP_D), row),
          pl.BlockSpec((64, 32), rep),
          pl.BlockSpec((1, 32), rep),
          pl.BlockSpec((32, 16), rep),
          pl.BlockSpec((1, 16), rep),
          pl.BlockSpec((32, 1), rep),
          pl.BlockSpec((1, 1), rep),
      ],
      out_specs=pl.BlockSpec((BLK, 1), row),
      out_shape=jax.ShapeDtypeStruct((B, 1), jnp.float32),
  )(ug, ig, um, im, w0, b0, w1, b1, wfc, bfc)


@jax.jit
def kernel(u, i, Ugmf, Igmf, Umlp, Imlp, W0, b0, W1, b1, Wfc, bfc):
  u2 = u.astype(jnp.int32).reshape(NW * NCHUNK, IDX_CHUNK)
  i2 = i.astype(jnp.int32).reshape(NW * NCHUNK, IDX_CHUNK)
  ug, ig, um, im = _sc_gather(u2, i2, Ugmf, Igmf, Umlp, Imlp)
  return _tc_mlp(ug, ig, um, im,
                 W0, b0.reshape(1, -1), W1, b1.reshape(1, -1),
                 Wfc, bfc.reshape(1, 1))
